# baseline (speedup 1.0000x reference)
import functools

import jax
import jax.numpy as jnp
from jax.experimental import pallas as pl
from jax.experimental.pallas import tpu as pltpu

D_MODEL = 2048
D_FF = 8192
M_BLK = 1024
F_BLK = 512
LN_EPS = 1e-5


def _ffn_kernel(x_ref, gamma_ref, beta_ref, w1_ref, b1_ref, w2_ref, b2_ref,
                out_ref, ln_scratch):
    f = pl.program_id(1)

    @pl.when(f == 0)
    def _init():
        x = x_ref[:]
        mu = jnp.mean(x, axis=-1, keepdims=True)
        var = jnp.mean((x - mu) ** 2, axis=-1, keepdims=True)
        ln = (x - mu) / jnp.sqrt(var + LN_EPS) * gamma_ref[0, :] + beta_ref[0, :]
        ln_scratch[:] = ln.astype(jnp.bfloat16)
        out_ref[:] = x + b2_ref[0, :]

    h = jnp.maximum(
        jnp.dot(ln_scratch[:], w1_ref[:], preferred_element_type=jnp.float32)
        + b1_ref[0, :],
        0.0,
    ).astype(jnp.bfloat16)
    out_ref[:] += jnp.dot(h, w2_ref[:], preferred_element_type=jnp.float32)


@jax.jit
def _run(x, gamma, beta, W1, b1, W2, b2):
    n = x.shape[0]
    grid = (n // M_BLK, D_FF // F_BLK)
    return pl.pallas_call(
        _ffn_kernel,
        grid=grid,
        in_specs=[
            pl.BlockSpec((M_BLK, D_MODEL), lambda m, f: (m, 0)),
            pl.BlockSpec((1, D_MODEL), lambda m, f: (0, 0)),
            pl.BlockSpec((1, D_MODEL), lambda m, f: (0, 0)),
            pl.BlockSpec((D_MODEL, F_BLK), lambda m, f: (0, f)),
            pl.BlockSpec((1, F_BLK), lambda m, f: (0, f)),
            pl.BlockSpec((F_BLK, D_MODEL), lambda m, f: (f, 0)),
            pl.BlockSpec((1, D_MODEL), lambda m, f: (0, 0)),
        ],
        out_specs=pl.BlockSpec((M_BLK, D_MODEL), lambda m, f: (m, 0)),
        out_shape=jax.ShapeDtypeStruct((n, D_MODEL), jnp.float32),
        scratch_shapes=[pltpu.VMEM((M_BLK, D_MODEL), jnp.bfloat16)],
    )(x, gamma, beta, W1, b1, W2, b2)


def kernel(input_features, expert_centroids, ln_gamma, ln_beta, W1, b1, W2, b2):
    d = input_features.shape[-1]
    x = input_features.reshape(-1, d)
    out = _run(
        x,
        ln_gamma.reshape(1, -1),
        ln_beta.reshape(1, -1),
        W1.astype(jnp.bfloat16),
        b1.reshape(1, -1),
        W2.astype(jnp.bfloat16),
        b2.reshape(1, -1),
    )
    return out.reshape(input_features.shape)


# speedup vs baseline: 1.0286x; 1.0286x over previous
import functools

import jax
import jax.numpy as jnp
from jax.experimental import pallas as pl
from jax.experimental.pallas import tpu as pltpu

D_MODEL = 2048
D_FF = 8192
M_BLK = 512
F_BLK = 2048
LN_EPS = 1e-5


def _ffn_kernel(x_ref, gamma_ref, beta_ref, w1_ref, b1_ref, w2_ref, b2_ref,
                out_ref, ln_scratch):
    f = pl.program_id(1)

    @pl.when(f == 0)
    def _init():
        x = x_ref[:]
        mu = jnp.mean(x, axis=-1, keepdims=True)
        var = jnp.mean((x - mu) ** 2, axis=-1, keepdims=True)
        ln = (x - mu) / jnp.sqrt(var + LN_EPS) * gamma_ref[0, :] + beta_ref[0, :]
        ln_scratch[:] = ln.astype(jnp.bfloat16)
        out_ref[:] = x + b2_ref[0, :]

    h = jnp.maximum(
        jnp.dot(ln_scratch[:], w1_ref[:], preferred_element_type=jnp.float32)
        + b1_ref[0, :],
        0.0,
    ).astype(jnp.bfloat16)
    out_ref[:] += jnp.dot(h, w2_ref[:], preferred_element_type=jnp.float32)


@jax.jit
def _run(x, gamma, beta, W1, b1, W2, b2):
    n = x.shape[0]
    grid = (n // M_BLK, D_FF // F_BLK)
    return pl.pallas_call(
        _ffn_kernel,
        grid=grid,
        in_specs=[
            pl.BlockSpec((M_BLK, D_MODEL), lambda m, f: (m, 0)),
            pl.BlockSpec((1, D_MODEL), lambda m, f: (0, 0)),
            pl.BlockSpec((1, D_MODEL), lambda m, f: (0, 0)),
            pl.BlockSpec((D_MODEL, F_BLK), lambda m, f: (0, f)),
            pl.BlockSpec((1, F_BLK), lambda m, f: (0, f)),
            pl.BlockSpec((F_BLK, D_MODEL), lambda m, f: (f, 0)),
            pl.BlockSpec((1, D_MODEL), lambda m, f: (0, 0)),
        ],
        out_specs=pl.BlockSpec((M_BLK, D_MODEL), lambda m, f: (m, 0)),
        out_shape=jax.ShapeDtypeStruct((n, D_MODEL), jnp.float32),
        scratch_shapes=[pltpu.VMEM((M_BLK, D_MODEL), jnp.bfloat16)],
    )(x, gamma, beta, W1, b1, W2, b2)


def kernel(input_features, expert_centroids, ln_gamma, ln_beta, W1, b1, W2, b2):
    d = input_features.shape[-1]
    x = input_features.reshape(-1, d)
    out = _run(
        x,
        ln_gamma.reshape(1, -1),
        ln_beta.reshape(1, -1),
        W1.astype(jnp.bfloat16),
        b1.reshape(1, -1),
        W2.astype(jnp.bfloat16),
        b2.reshape(1, -1),
    )
    return out.reshape(input_features.shape)
